# Initial kernel scaffold; baseline (speedup 1.0000x reference)
import jax
import jax.numpy as jnp
from jax.experimental import pallas as pl


def kernel(x, edge_index, edge_weight, kernel, bias):
    raise NotImplementedError("write your pallas kernel here")


# speedup vs baseline: 6.2021x; 6.2021x over previous
import functools

import jax
import jax.numpy as jnp
from jax import lax
from jax.experimental import pallas as pl
from jax.experimental.pallas import tpu as pltpu
from jax.experimental.pallas import tpu_sc as plsc

N = 10000
E = 320000
D = 128
K = 3

NC = 2
NS = 16
NW = NC * NS
L = 16

N_PAD = 10240
STRIPE = N_PAD // NS
C = 80
EPW = E // NW
NF = D // L

_mesh = plsc.VectorSubcoreMesh(
    core_axis_name="c", subcore_axis_name="s", num_cores=NC, num_subcores=NS)


def _zero_rows(buf, nrows):
  def row(r, _):
    for f in range(NF):
      buf[r, pl.ds(f * L, L)] = jnp.zeros((L,), jnp.float32)
    return 0
  lax.fori_loop(0, nrows, row, 0)


def _deg_body(row_hbm, ew_hbm, degp_hbm, idx_b, val_b, line_b, acc):
  c = lax.axis_index("c")
  s = lax.axis_index("s")
  wid = s * NC + c
  base = s * STRIPE
  def zrow(r, _):
    line_b[pl.ds(r * L, L)] = jnp.zeros((L,), jnp.float32)
    return 0
  lax.fori_loop(0, STRIPE // L, zrow, 0)
  pltpu.sync_copy(line_b, acc.at[pl.ds(base, STRIPE)])
  plsc.subcore_barrier()
  ebase = wid * EPW
  def chunk(i, _):
    off = ebase + i * C
    pltpu.sync_copy(row_hbm.at[pl.ds(off, C)], idx_b)
    pltpu.sync_copy(ew_hbm.at[pl.ds(off, C)], val_b)
    pltpu.sync_copy(val_b, acc.at[idx_b], add=True)
    return 0
  lax.fori_loop(0, EPW // C, chunk, 0)
  plsc.subcore_barrier()
  pltpu.sync_copy(acc.at[pl.ds(base, STRIPE)], line_b)
  pltpu.sync_copy(line_b, degp_hbm.at[c, pl.ds(base, STRIPE)])


_deg_kernel = pl.kernel(
    _deg_body,
    out_type=jax.ShapeDtypeStruct((NC, N_PAD), jnp.float32),
    mesh=_mesh,
    compiler_params=pltpu.CompilerParams(needs_layout_passes=False),
    scratch_types=[
        pltpu.VMEM((C,), jnp.int32),
        pltpu.VMEM((C,), jnp.float32),
        pltpu.VMEM((STRIPE,), jnp.float32),
        pltpu.VMEM_SHARED((N_PAD,), jnp.float32),
    ],
)


def _dinv_body(degp_ref, dinv_ref):
  deg = degp_ref[0, :] + degp_ref[1, :]
  dinv_ref[...] = jnp.where(deg > 0, lax.rsqrt(jnp.where(deg > 0, deg, 1.0)),
                            jnp.zeros_like(deg))


def _dinv_tc(degp):
  return pl.pallas_call(
      _dinv_body,
      out_shape=jax.ShapeDtypeStruct((N_PAD,), jnp.float32),
  )(degp)


def _normw_body(row_hbm, col_hbm, ew_hbm, dinv_hbm, w2_hbm,
                dinv_b, r_b, c_b, ew_b, out_b):
  c = lax.axis_index("c")
  s = lax.axis_index("s")
  wid = s * NC + c
  pltpu.sync_copy(dinv_hbm, dinv_b)
  ebase = wid * EPW
  def chunk(i, _):
    off = ebase + i * C
    pltpu.sync_copy(row_hbm.at[pl.ds(off, C)], r_b)
    pltpu.sync_copy(col_hbm.at[pl.ds(off, C)], c_b)
    pltpu.sync_copy(ew_hbm.at[pl.ds(off, C)], ew_b)
    for v in range(C // L):
      r16 = r_b[pl.ds(v * L, L)]
      c16 = c_b[pl.ds(v * L, L)]
      dr = plsc.load_gather(dinv_b, [r16])
      dc = plsc.load_gather(dinv_b, [c16])
      out_b[pl.ds(v * L, L)] = dr * ew_b[pl.ds(v * L, L)] * dc
    pltpu.sync_copy(out_b, w2_hbm.at[pl.ds(off, C)])
    return 0
  lax.fori_loop(0, EPW // C, chunk, 0)


_normw_kernel = pl.kernel(
    _normw_body,
    out_type=jax.ShapeDtypeStruct((E,), jnp.float32),
    mesh=_mesh,
    compiler_params=pltpu.CompilerParams(needs_layout_passes=False),
    scratch_types=[
        pltpu.VMEM((N_PAD,), jnp.float32),
        pltpu.VMEM((C,), jnp.int32),
        pltpu.VMEM((C,), jnp.int32),
        pltpu.VMEM((C,), jnp.float32),
        pltpu.VMEM((C,), jnp.float32),
    ],
)


def _hop_body(h_hbm, row_hbm, col_hbm, w2_hbm, pout_hbm,
              idx_r, idx_c, w_b, rows_b, bounce, acc):
  c = lax.axis_index("c")
  s = lax.axis_index("s")
  wid = s * NC + c
  base = s * STRIPE

  @pl.when(c == 0)
  def _():
    def ij(j, _):
      pltpu.sync_copy(h_hbm.at[pl.ds(base + j * C, C), :], bounce)
      pltpu.sync_copy(bounce, acc.at[pl.ds(base + j * C, C), :])
      return 0
    lax.fori_loop(0, STRIPE // C, ij, 0)

  @pl.when(c == 1)
  def _():
    _zero_rows(bounce, C)
    def zj(j, _):
      pltpu.sync_copy(bounce, acc.at[pl.ds(base + j * C, C), :])
      return 0
    lax.fori_loop(0, STRIPE // C, zj, 0)

  plsc.subcore_barrier()

  ebase = wid * EPW
  def chunk(i, _):
    off = ebase + i * C
    pltpu.sync_copy(row_hbm.at[pl.ds(off, C)], idx_r)
    pltpu.sync_copy(col_hbm.at[pl.ds(off, C)], idx_c)
    pltpu.sync_copy(w2_hbm.at[pl.ds(off, C)], w_b)
    pltpu.sync_copy(h_hbm.at[idx_r], rows_b)
    def scale(e, _):
      wv = plsc.load_gather(w_b, [jnp.full((L,), e, jnp.int32)])
      for f in range(NF):
        rows_b[e, pl.ds(f * L, L)] = rows_b[e, pl.ds(f * L, L)] * wv
      return 0
    lax.fori_loop(0, C, scale, 0)
    pltpu.sync_copy(rows_b, acc.at[idx_c], add=True)
    return 0
  lax.fori_loop(0, EPW // C, chunk, 0)

  plsc.subcore_barrier()

  def oj(j, _):
    pltpu.sync_copy(acc.at[pl.ds(base + j * C, C), :], bounce)
    pltpu.sync_copy(bounce, pout_hbm.at[c, pl.ds(base + j * C, C), :])
    return 0
  lax.fori_loop(0, STRIPE // C, oj, 0)


_hop_kernel = pl.kernel(
    _hop_body,
    out_type=jax.ShapeDtypeStruct((NC, N_PAD, D), jnp.float32),
    mesh=_mesh,
    compiler_params=pltpu.CompilerParams(needs_layout_passes=False),
    scratch_types=[
        pltpu.VMEM((C,), jnp.int32),
        pltpu.VMEM((C,), jnp.int32),
        pltpu.VMEM((C,), jnp.float32),
        pltpu.VMEM((C, D), jnp.float32),
        pltpu.VMEM((C, D), jnp.float32),
        pltpu.VMEM_SHARED((N_PAD, D), jnp.float32),
    ],
)


def _comb_body(p_ref, h_ref):
  h_ref[...] = p_ref[0] + p_ref[1]


def _combine_tc(p):
  blk = 1024
  return pl.pallas_call(
      _comb_body,
      grid=(N_PAD // blk,),
      in_specs=[pl.BlockSpec((NC, blk, D), lambda i: (0, i, 0))],
      out_specs=pl.BlockSpec((blk, D), lambda i: (i, 0)),
      out_shape=jax.ShapeDtypeStruct((N_PAD, D), jnp.float32),
  )(p)


def _mm_body(x_ref, h1_ref, h2_ref, p3_ref, w_ref, b_ref, o_ref):
  h3 = p3_ref[0] + p3_ref[1]
  acc = jnp.dot(x_ref[...], w_ref[pl.ds(0, D), :],
                preferred_element_type=jnp.float32)
  acc += jnp.dot(h1_ref[...], w_ref[pl.ds(D, D), :],
                 preferred_element_type=jnp.float32)
  acc += jnp.dot(h2_ref[...], w_ref[pl.ds(2 * D, D), :],
                 preferred_element_type=jnp.float32)
  acc += jnp.dot(h3, w_ref[pl.ds(3 * D, D), :],
                 preferred_element_type=jnp.float32)
  o_ref[...] = acc + b_ref[...]


def _matmul_tc(x, h1, h2, p3, w, b):
  blk = 400
  grid = N // blk
  return pl.pallas_call(
      _mm_body,
      grid=(grid,),
      in_specs=[
          pl.BlockSpec((blk, D), lambda i: (i, 0)),
          pl.BlockSpec((blk, D), lambda i: (i, 0)),
          pl.BlockSpec((blk, D), lambda i: (i, 0)),
          pl.BlockSpec((NC, blk, D), lambda i: (0, i, 0)),
          pl.BlockSpec(((K + 1) * D, D), lambda i: (0, 0)),
          pl.BlockSpec((1, D), lambda i: (0, 0)),
      ],
      out_specs=pl.BlockSpec((blk, D), lambda i: (i, 0)),
      out_shape=jax.ShapeDtypeStruct((N, D), jnp.float32),
  )(x, h1, h2, p3, w, b)


def kernel(x, edge_index, edge_weight, kernel, bias):
  w = kernel
  row = edge_index[0]
  col = edge_index[1]
  x_pad = jnp.zeros((N_PAD, D), jnp.float32).at[:N].set(x)

  degp = _deg_kernel(row, edge_weight)
  dinv = _dinv_tc(degp)
  w2 = _normw_kernel(row, col, edge_weight, dinv)

  p1 = _hop_kernel(x_pad, row, col, w2)
  h1 = _combine_tc(p1)
  p2 = _hop_kernel(h1, row, col, w2)
  h2 = _combine_tc(p2)
  p3 = _hop_kernel(h2, row, col, w2)

  out = _matmul_tc(x_pad[:N], h1[:N], h2[:N], p3[:, :N], w,
                   bias.reshape(1, D))
  return out
